# baseline (speedup 1.0000x reference)
import jax
import jax.numpy as jnp
from jax import lax
from jax.experimental import pallas as pl
from jax.experimental.pallas import tpu as pltpu

NUM_EMBEDDINGS = 1024
EMBEDDING_DIM = 64
COMMITMENT_COST = 0.25
BATCH = 16
TOKENS = 576

N = BATCH * TOKENS
T = 512
NB = N // T


def _vq_block(x_ref, w_ref, qst_ref, idx_ref, loss_ref):
    i = pl.program_id(0)
    x = x_ref[...]
    w = w_ref[...]

    wn = w / jnp.maximum(
        jnp.sqrt(jnp.sum(w * w, axis=1, keepdims=True)), 1e-12)
    xn = x / jnp.maximum(
        jnp.sqrt(jnp.sum(x * x, axis=1, keepdims=True)), 1e-12)

    scores = lax.dot_general(xn, wn, (((1,), (1,)), ((), ())),
                             preferred_element_type=jnp.float32)
    d = 1.0 - scores
    idx = jnp.argmin(d, axis=1).astype(jnp.int32)

    onehot = (lax.broadcasted_iota(jnp.int32, (T, NUM_EMBEDDINGS), 1)
              == idx[:, None]).astype(jnp.float32)
    q = lax.dot_general(onehot, wn, (((1,), (0,)), ((), ())),
                        preferred_element_type=jnp.float32)

    qst_ref[...] = x + (q - x)
    idx_ref[0, 0, :] = idx

    part = jnp.sum((q - x) ** 2)

    @pl.when(i == 0)
    def _():
        loss_ref[0, 0] = 0.0

    loss_ref[0, 0] += part

    @pl.when(i == NB - 1)
    def _():
        loss_ref[0, 0] = loss_ref[0, 0] * (COMMITMENT_COST / (N * EMBEDDING_DIM))


def kernel(inputs, W):
    flat = inputs.reshape(N, EMBEDDING_DIM)
    qst, idx, loss = pl.pallas_call(
        _vq_block,
        grid=(NB,),
        in_specs=[
            pl.BlockSpec((T, EMBEDDING_DIM), lambda i: (i, 0)),
            pl.BlockSpec((NUM_EMBEDDINGS, EMBEDDING_DIM), lambda i: (0, 0)),
        ],
        out_specs=[
            pl.BlockSpec((T, EMBEDDING_DIM), lambda i: (i, 0)),
            pl.BlockSpec((1, 1, T), lambda i: (i, 0, 0)),
            pl.BlockSpec(memory_space=pltpu.SMEM, block_shape=(1, 1),
                         index_map=lambda i: (0, 0)),
        ],
        out_shape=[
            jax.ShapeDtypeStruct((N, EMBEDDING_DIM), jnp.float32),
            jax.ShapeDtypeStruct((NB, 1, T), jnp.int32),
            jax.ShapeDtypeStruct((1, 1), jnp.float32),
        ],
    )(flat, W)
    return (qst.reshape(inputs.shape), loss[0, 0],
            idx.reshape(BATCH, TOKENS))


# speedup vs baseline: 1.0506x; 1.0506x over previous
import jax
import jax.numpy as jnp
from jax import lax
from jax.experimental import pallas as pl
from jax.experimental.pallas import tpu as pltpu

NUM_EMBEDDINGS = 1024
EMBEDDING_DIM = 64
COMMITMENT_COST = 0.25
BATCH = 16
TOKENS = 576

N = BATCH * TOKENS
T = 512
NB = N // T


def _vq_block(x_ref, w_ref, qst_ref, idx_ref, loss_ref, wn_ref):
    i = pl.program_id(0)
    x = x_ref[...]

    @pl.when(i == 0)
    def _():
        w = w_ref[...]
        wn_ref[...] = w / jnp.maximum(
            jnp.sqrt(jnp.sum(w * w, axis=1, keepdims=True)), 1e-12)

    wn = wn_ref[...]
    xn = x / jnp.maximum(
        jnp.sqrt(jnp.sum(x * x, axis=1, keepdims=True)), 1e-12)

    scores = lax.dot_general(xn, wn, (((1,), (1,)), ((), ())),
                             preferred_element_type=jnp.float32)
    d = 1.0 - scores
    idx = jnp.argmin(d, axis=1).astype(jnp.int32)

    onehot = (lax.broadcasted_iota(jnp.int32, (T, NUM_EMBEDDINGS), 1)
              == idx[:, None]).astype(jnp.float32)
    q = lax.dot_general(onehot, wn, (((1,), (0,)), ((), ())),
                        preferred_element_type=jnp.float32)

    qst_ref[...] = x + (q - x)
    idx_ref[0, 0, :] = idx

    part = jnp.sum((q - x) ** 2)

    @pl.when(i == 0)
    def _():
        loss_ref[0, 0] = 0.0

    loss_ref[0, 0] += part

    @pl.when(i == NB - 1)
    def _():
        loss_ref[0, 0] = loss_ref[0, 0] * (COMMITMENT_COST / (N * EMBEDDING_DIM))


def kernel(inputs, W):
    flat = inputs.reshape(N, EMBEDDING_DIM)
    qst, idx, loss = pl.pallas_call(
        _vq_block,
        grid=(NB,),
        in_specs=[
            pl.BlockSpec((T, EMBEDDING_DIM), lambda i: (i, 0)),
            pl.BlockSpec((NUM_EMBEDDINGS, EMBEDDING_DIM), lambda i: (0, 0)),
        ],
        out_specs=[
            pl.BlockSpec((T, EMBEDDING_DIM), lambda i: (i, 0)),
            pl.BlockSpec((1, 1, T), lambda i: (i, 0, 0)),
            pl.BlockSpec(memory_space=pltpu.SMEM, block_shape=(1, 1),
                         index_map=lambda i: (0, 0)),
        ],
        out_shape=[
            jax.ShapeDtypeStruct((N, EMBEDDING_DIM), jnp.float32),
            jax.ShapeDtypeStruct((NB, 1, T), jnp.int32),
            jax.ShapeDtypeStruct((1, 1), jnp.float32),
        ],
        scratch_shapes=[
            pltpu.VMEM((NUM_EMBEDDINGS, EMBEDDING_DIM), jnp.float32),
        ],
    )(flat, W)
    return (qst.reshape(inputs.shape), loss[0, 0],
            idx.reshape(BATCH, TOKENS))
